# baseline (speedup 1.0000x reference)
import jax
import jax.numpy as jnp
from jax import lax
from jax.experimental import pallas as pl
from jax.experimental.pallas import tpu as pltpu
from jax.experimental.pallas import tpu_sc as plsc

BATCH = 4096
SEQ = 200
EMBED_DIM = 32
VOCAB = 1000000
VP = 1000064

NC = 2
NS = 16
NW = NC * NS

TOTAL = BATCH * SEQ

AQ = 512
AQ_PER_W = 62
A_COL_MAX = 999424
A_TAIL = 999936

BPW = TOTAL // NW
S = 128
NSTR = BPW // S
G = 5
GROUPS = NSTR // G
GR = G * S
R = 4
Q = GROUPS // R

CCH = 512
NCH = TOTAL // CCH
CCH_PER_W = NCH // NW


def _iota16():
    return lax.iota(jnp.int32, 16)


def _a_body(embT_hbm, tail_hbm, out_hbm, in_a, in_b, o_a, o_b, isem_a, isem_b, osem_a, osem_b):
    c = lax.axis_index("c")
    s = lax.axis_index("s")
    wid = s * NC + c

    def col0_of(t):
        return pl.multiple_of(jnp.minimum((wid + NW * t) * AQ, A_COL_MAX), AQ)

    def issue_in(t, buf, sem):
        col0 = col0_of(t)
        for i in range(4):
            pltpu.async_copy(
                embT_hbm.at[pl.ds(8 * i, 8), pl.ds(col0, AQ)],
                buf.at[pl.ds(8 * i, 8), pl.ds(0, AQ)],
                sem,
            )

    def wait_in(buf, sem):
        for i in range(4):
            pltpu.make_async_copy(
                embT_hbm.at[pl.ds(8 * i, 8), pl.ds(0, AQ)],
                buf.at[pl.ds(8 * i, 8), pl.ds(0, AQ)],
                sem,
            ).wait()

    def transpose(ibuf, obuf):
        @plsc.parallel_loop(0, AQ // 8, unroll=2)
        def _(ib):
            bvec = jnp.full((16,), 0, jnp.int32) + ib * 8
            row0 = ib * 2
            for k in range(8):
                for c16 in range(2):
                    rows = _iota16() + (c16 * 16)
                    vec = plsc.load_gather(ibuf, [rows, bvec + k])
                    obuf[row0 + k // 4,
                         pl.ds((k % 4) * 32 + c16 * 16, 16)] = vec

    def issue_out(t, buf, sem):
        vr0 = pl.multiple_of(col0_of(t) // 4, 128)
        pltpu.async_copy(buf, out_hbm.at[pl.ds(vr0, 128), :], sem)

    def wait_out(buf, sem):
        pltpu.make_async_copy(buf, out_hbm.at[pl.ds(0, 128), :], sem).wait()

    issue_in(0, in_a, isem_a)
    issue_in(1, in_b, isem_b)

    def round_(u, carry):
        for t, ibuf, obuf, isem, osem in (
            (2 * u, in_a, o_a, isem_a, osem_a),
            (2 * u + 1, in_b, o_b, isem_b, osem_b),
        ):
            wait_in(ibuf, isem)

            @pl.when(u > 0)
            def _():
                wait_out(obuf, osem)

            transpose(ibuf, obuf)
            issue_out(t, obuf, osem)

            @pl.when(t + 2 < AQ_PER_W)
            def _():
                issue_in(t + 2, ibuf, isem)

        return carry

    lax.fori_loop(0, AQ_PER_W // 2, round_, 0)
    wait_out(o_a, osem_a)
    wait_out(o_b, osem_b)

    @pl.when(wid == 0)
    def _():
        pltpu.async_copy(tail_hbm, o_a.at[pl.ds(0, 16), :], isem_a).wait()
        pltpu.async_copy(
            o_a.at[pl.ds(0, 16), :],
            out_hbm.at[pl.ds(A_TAIL // 4, 16), :],
            osem_a,
        ).wait()


def _b_body(x_hbm, tab_hbm, out_hbm, idx_v, bufs, gsems, osems):
    c = lax.axis_index("c")
    s = lax.axis_index("s")
    wid = s * NC + c
    base = wid * BPW

    pltpu.sync_copy(x_hbm.at[wid], idx_v)

    def issue_gathers(g, b):
        for j in range(G):
            pltpu.async_copy(
                tab_hbm.at[idx_v.at[g * G + j]],
                bufs[b].at[pl.ds(j * S, S)],
                gsems[b],
            )

    def wait_gathers(b):
        for j in range(G):
            pltpu.make_async_copy(
                tab_hbm.at[idx_v.at[j]],
                bufs[b].at[pl.ds(j * S, S)],
                gsems[b],
            ).wait()

    def issue_outcopy(g, b):
        pltpu.async_copy(bufs[b], out_hbm.at[pl.ds(base + g * GR, GR)], osems[b])

    def wait_outcopy(b):
        pltpu.make_async_copy(
            bufs[b], out_hbm.at[pl.ds(base, GR)], osems[b]
        ).wait()

    for b in range(R):
        issue_gathers(b, b)

    def ring_round(q, carry):
        for b in range(R):
            wait_gathers(b)
            issue_outcopy(q * R + b, b)
        for b in range(R):
            wait_outcopy(b)
            issue_gathers((q + 1) * R + b, b)
        return carry

    lax.fori_loop(0, Q - 1, ring_round, 0)

    for b in range(R):
        wait_gathers(b)
        issue_outcopy((Q - 1) * R + b, b)
    for b in range(R):
        wait_outcopy(b)


def _c_body(rows_hbm, out_hbm, v_a, v_b, o_a, o_b, isem_a, isem_b, osem_a, osem_b):
    c = lax.axis_index("c")
    s = lax.axis_index("s")
    wid = s * NC + c
    g0 = wid * CCH_PER_W
    def issue_in(g, buf, sem):
        pltpu.async_copy(
            rows_hbm.at[pl.ds(pl.multiple_of(g * 128, 128), 128), :], buf, sem
        )

    def wait_in(buf, sem):
        pltpu.make_async_copy(rows_hbm.at[pl.ds(0, 128), :], buf, sem).wait()

    def transpose(vbuf, obuf):
        @plsc.parallel_loop(0, 128, unroll=4)
        def _(vr):
            bvec = jnp.full((16,), 0, jnp.int32) + vr * 4
            for q in range(4):
                for h in range(2):
                    vec = vbuf[vr, pl.ds(q * 32 + h * 16, 16)]
                    plsc.store_scatter(obuf, [_iota16() + h * 16, bvec + q], vec)

    def issue_out(g, buf, sem):
        sl = g // 8
        jb = pl.multiple_of((g % 8) * CCH, CCH)
        for i in range(4):
            pltpu.async_copy(
                buf.at[pl.ds(8 * i, 8), pl.ds(0, CCH)],
                out_hbm.at[sl, pl.ds(8 * i, 8), pl.ds(jb, CCH)],
                sem,
            )

    def wait_out(buf, sem):
        for i in range(4):
            pltpu.make_async_copy(
                buf.at[pl.ds(8 * i, 8), pl.ds(0, CCH)],
                out_hbm.at[0, pl.ds(8 * i, 8), pl.ds(0, CCH)],
                sem,
            ).wait()

    issue_in(g0, v_a, isem_a)
    issue_in(g0 + 1, v_b, isem_b)

    def round_(u, carry):
        for off, vbuf, obuf, isem, osem in (
            (0, v_a, o_a, isem_a, osem_a),
            (1, v_b, o_b, isem_b, osem_b),
        ):
            g = g0 + 2 * u + off
            wait_in(vbuf, isem)

            @pl.when(u > 0)
            def _():
                wait_out(obuf, osem)

            transpose(vbuf, obuf)
            issue_out(g, obuf, osem)

            @pl.when(2 * u + off + 2 < CCH_PER_W)
            def _():
                issue_in(g + 2, vbuf, isem)

        return carry

    lax.fori_loop(0, CCH_PER_W // 2, round_, 0)
    wait_out(o_a, osem_a)
    wait_out(o_b, osem_b)


def kernel(x, mask, embedding_matrix):
    del mask
    mesh = plsc.VectorSubcoreMesh(core_axis_name="c", subcore_axis_name="s")

    embT = jnp.transpose(embedding_matrix)

    a = pl.kernel(
        _a_body,
        out_type=jax.ShapeDtypeStruct((VP // 4, 128), jnp.float32),
        mesh=mesh,
        compiler_params=pltpu.CompilerParams(use_tc_tiling_on_sc=True, needs_layout_passes=False, disable_bounds_checks=True),
        scratch_types=[
            pltpu.VMEM((32, AQ + 16), jnp.float32),
            pltpu.VMEM((32, AQ + 16), jnp.float32),
            pltpu.VMEM((128, 128), jnp.float32),
            pltpu.VMEM((128, 128), jnp.float32),
            pltpu.SemaphoreType.DMA,
            pltpu.SemaphoreType.DMA,
            pltpu.SemaphoreType.DMA,
            pltpu.SemaphoreType.DMA,
        ],
    )
    tail = lax.slice(embedding_matrix, (A_TAIL, 0), (VOCAB, EMBED_DIM))
    tail = jnp.reshape(tail, (16, 128))
    tab = jnp.reshape(a(embT, tail), (VP, EMBED_DIM))

    xr = jnp.transpose(x).astype(jnp.int32).reshape(NW, NSTR, S)
    b = pl.kernel(
        _b_body,
        out_type=jax.ShapeDtypeStruct((TOTAL, EMBED_DIM), jnp.float32),
        mesh=mesh,
        compiler_params=pltpu.CompilerParams(use_tc_tiling_on_sc=False),
        scratch_types=[
            pltpu.VMEM((NSTR, S), jnp.int32),
            [pltpu.VMEM((GR, EMBED_DIM), jnp.float32) for _ in range(R)],
            [pltpu.SemaphoreType.DMA for _ in range(R)],
            [pltpu.SemaphoreType.DMA for _ in range(R)],
        ],
    )
    rows = b(xr, tab)

    rows2 = jnp.reshape(rows, (TOTAL * EMBED_DIM // 128, 128))
    cker = pl.kernel(
        _c_body,
        out_type=jax.ShapeDtypeStruct((SEQ, EMBED_DIM, BATCH), jnp.float32),
        mesh=mesh,
        compiler_params=pltpu.CompilerParams(use_tc_tiling_on_sc=True, needs_layout_passes=False, disable_bounds_checks=True),
        scratch_types=[
            pltpu.VMEM((128, 128), jnp.float32),
            pltpu.VMEM((128, 128), jnp.float32),
            pltpu.VMEM((EMBED_DIM, CCH + 16), jnp.float32),
            pltpu.VMEM((EMBED_DIM, CCH + 16), jnp.float32),
            pltpu.SemaphoreType.DMA,
            pltpu.SemaphoreType.DMA,
            pltpu.SemaphoreType.DMA,
            pltpu.SemaphoreType.DMA,
        ],
    )
    outp = cker(rows2)
    return jnp.transpose(outp, (2, 0, 1))


# speedup vs baseline: 1.0894x; 1.0894x over previous
import jax
import jax.numpy as jnp
from jax import lax
from jax.experimental import pallas as pl
from jax.experimental.pallas import tpu as pltpu
from jax.experimental.pallas import tpu_sc as plsc

BATCH = 4096
SEQ = 200
EMBED_DIM = 32
VOCAB = 1000000
VP = 1000064

NC = 2
NS = 16
NW = NC * NS

TOTAL = BATCH * SEQ

AQ = 512
AQ_PER_W = 62
A_COL_MAX = 999424
A_TAIL = 999936

BPW = TOTAL // NW
S = 128
NSTR = BPW // S
G = 5
GROUPS = NSTR // G
GR = G * S
R = 4
Q = GROUPS // R

CCH = 512
NCH = TOTAL // CCH
CCH_PER_W = NCH // NW


def _iota16():
    return lax.iota(jnp.int32, 16)


def _a_body(embT_hbm, tail_hbm, out_hbm, in_a, in_b, o_a, o_b, isem_a, isem_b, osem_a, osem_b):
    c = lax.axis_index("c")
    s = lax.axis_index("s")
    wid = s * NC + c

    def col0_of(t):
        return pl.multiple_of(jnp.minimum((wid + NW * t) * AQ, A_COL_MAX), AQ)

    def issue_in(t, buf, sem):
        col0 = col0_of(t)
        for i in range(4):
            pltpu.async_copy(
                embT_hbm.at[pl.ds(8 * i, 8), pl.ds(col0, AQ)],
                buf.at[pl.ds(8 * i, 8), pl.ds(0, AQ)],
                sem,
            )

    def wait_in(buf, sem):
        for i in range(4):
            pltpu.make_async_copy(
                embT_hbm.at[pl.ds(8 * i, 8), pl.ds(0, AQ)],
                buf.at[pl.ds(8 * i, 8), pl.ds(0, AQ)],
                sem,
            ).wait()

    def transpose(ibuf, obuf):
        @plsc.parallel_loop(0, AQ, unroll=16)
        def _(i):
            row = i // 4
            colbase = (i % 4) * 32
            for c16 in range(2):
                rows = _iota16() + (c16 * 16)
                cols = jnp.full((16,), 0, jnp.int32) + i
                vec = plsc.load_gather(ibuf, [rows, cols])
                obuf[row, pl.ds(colbase + c16 * 16, 16)] = vec

    def issue_out(t, buf, sem):
        vr0 = pl.multiple_of(col0_of(t) // 4, 128)
        pltpu.async_copy(buf, out_hbm.at[pl.ds(vr0, 128), :], sem)

    def wait_out(buf, sem):
        pltpu.make_async_copy(buf, out_hbm.at[pl.ds(0, 128), :], sem).wait()

    issue_in(0, in_a, isem_a)
    issue_in(1, in_b, isem_b)

    def round_(u, carry):
        for t, ibuf, obuf, isem, osem in (
            (2 * u, in_a, o_a, isem_a, osem_a),
            (2 * u + 1, in_b, o_b, isem_b, osem_b),
        ):
            wait_in(ibuf, isem)

            @pl.when(u > 0)
            def _():
                wait_out(obuf, osem)

            transpose(ibuf, obuf)
            issue_out(t, obuf, osem)

            @pl.when(t + 2 < AQ_PER_W)
            def _():
                issue_in(t + 2, ibuf, isem)

        return carry

    lax.fori_loop(0, AQ_PER_W // 2, round_, 0)
    wait_out(o_a, osem_a)
    wait_out(o_b, osem_b)

    @pl.when(wid == 0)
    def _():
        pltpu.async_copy(tail_hbm, o_a.at[pl.ds(0, 16), :], isem_a).wait()
        pltpu.async_copy(
            o_a.at[pl.ds(0, 16), :],
            out_hbm.at[pl.ds(A_TAIL // 4, 16), :],
            osem_a,
        ).wait()


def _b_body(x_hbm, tab_hbm, out_hbm, idx_v, bufs, gsems, osems):
    c = lax.axis_index("c")
    s = lax.axis_index("s")
    wid = s * NC + c
    base = wid * BPW

    pltpu.sync_copy(x_hbm.at[wid], idx_v)

    def issue_gathers(g, b):
        for j in range(G):
            pltpu.async_copy(
                tab_hbm.at[idx_v.at[g * G + j]],
                bufs[b].at[pl.ds(j * S, S)],
                gsems[b],
            )

    def wait_gathers(b):
        for j in range(G):
            pltpu.make_async_copy(
                tab_hbm.at[idx_v.at[j]],
                bufs[b].at[pl.ds(j * S, S)],
                gsems[b],
            ).wait()

    def issue_outcopy(g, b):
        pltpu.async_copy(bufs[b], out_hbm.at[pl.ds(base + g * GR, GR)], osems[b])

    def wait_outcopy(b):
        pltpu.make_async_copy(
            bufs[b], out_hbm.at[pl.ds(base, GR)], osems[b]
        ).wait()

    for b in range(R):
        issue_gathers(b, b)

    def ring_round(q, carry):
        for b in range(R):
            wait_gathers(b)
            issue_outcopy(q * R + b, b)
        for b in range(R):
            wait_outcopy(b)
            issue_gathers((q + 1) * R + b, b)
        return carry

    lax.fori_loop(0, Q - 1, ring_round, 0)

    for b in range(R):
        wait_gathers(b)
        issue_outcopy((Q - 1) * R + b, b)
    for b in range(R):
        wait_outcopy(b)


def _c_body(rows_hbm, out_hbm, v_a, v_b, o_a, o_b, isem_a, isem_b, osem_a, osem_b):
    c = lax.axis_index("c")
    s = lax.axis_index("s")
    wid = s * NC + c
    g0 = wid * CCH_PER_W
    def issue_in(g, buf, sem):
        pltpu.async_copy(
            rows_hbm.at[pl.ds(pl.multiple_of(g * 128, 128), 128), :], buf, sem
        )

    def wait_in(buf, sem):
        pltpu.make_async_copy(rows_hbm.at[pl.ds(0, 128), :], buf, sem).wait()

    def transpose(vbuf, obuf):
        @plsc.parallel_loop(0, 128, unroll=8)
        def _(vr):
            for q in range(4):
                bloc = jnp.full((16,), 0, jnp.int32) + (vr * 4 + q)
                for h in range(2):
                    vec = vbuf[vr, pl.ds(q * 32 + h * 16, 16)]
                    plsc.store_scatter(obuf, [_iota16() + h * 16, bloc], vec)

    def issue_out(g, buf, sem):
        sl = g // 8
        jb = pl.multiple_of((g % 8) * CCH, CCH)
        for i in range(4):
            pltpu.async_copy(
                buf.at[pl.ds(8 * i, 8), pl.ds(0, CCH)],
                out_hbm.at[sl, pl.ds(8 * i, 8), pl.ds(jb, CCH)],
                sem,
            )

    def wait_out(buf, sem):
        for i in range(4):
            pltpu.make_async_copy(
                buf.at[pl.ds(8 * i, 8), pl.ds(0, CCH)],
                out_hbm.at[0, pl.ds(8 * i, 8), pl.ds(0, CCH)],
                sem,
            ).wait()

    issue_in(g0, v_a, isem_a)
    issue_in(g0 + 1, v_b, isem_b)

    def round_(u, carry):
        for off, vbuf, obuf, isem, osem in (
            (0, v_a, o_a, isem_a, osem_a),
            (1, v_b, o_b, isem_b, osem_b),
        ):
            g = g0 + 2 * u + off
            wait_in(vbuf, isem)

            @pl.when(u > 0)
            def _():
                wait_out(obuf, osem)

            transpose(vbuf, obuf)
            issue_out(g, obuf, osem)

            @pl.when(2 * u + off + 2 < CCH_PER_W)
            def _():
                issue_in(g + 2, vbuf, isem)

        return carry

    lax.fori_loop(0, CCH_PER_W // 2, round_, 0)
    wait_out(o_a, osem_a)
    wait_out(o_b, osem_b)


def kernel(x, mask, embedding_matrix):
    del mask
    mesh = plsc.VectorSubcoreMesh(core_axis_name="c", subcore_axis_name="s")

    embT = jnp.transpose(embedding_matrix)

    a = pl.kernel(
        _a_body,
        out_type=jax.ShapeDtypeStruct((VP // 4, 128), jnp.float32),
        mesh=mesh,
        compiler_params=pltpu.CompilerParams(use_tc_tiling_on_sc=True, needs_layout_passes=False, disable_bounds_checks=True),
        scratch_types=[
            pltpu.VMEM((32, AQ + 16), jnp.float32),
            pltpu.VMEM((32, AQ + 16), jnp.float32),
            pltpu.VMEM((128, 128), jnp.float32),
            pltpu.VMEM((128, 128), jnp.float32),
            pltpu.SemaphoreType.DMA,
            pltpu.SemaphoreType.DMA,
            pltpu.SemaphoreType.DMA,
            pltpu.SemaphoreType.DMA,
        ],
    )
    tail = lax.slice(embedding_matrix, (A_TAIL, 0), (VOCAB, EMBED_DIM))
    tail = jnp.reshape(tail, (16, 128))
    tab = jnp.reshape(a(embT, tail), (VP, EMBED_DIM))

    xr = jnp.transpose(x).astype(jnp.int32).reshape(NW, NSTR, S)
    b = pl.kernel(
        _b_body,
        out_type=jax.ShapeDtypeStruct((TOTAL, EMBED_DIM), jnp.float32),
        mesh=mesh,
        compiler_params=pltpu.CompilerParams(use_tc_tiling_on_sc=False),
        scratch_types=[
            pltpu.VMEM((NSTR, S), jnp.int32),
            [pltpu.VMEM((GR, EMBED_DIM), jnp.float32) for _ in range(R)],
            [pltpu.SemaphoreType.DMA for _ in range(R)],
            [pltpu.SemaphoreType.DMA for _ in range(R)],
        ],
    )
    rows = b(xr, tab)

    rows2 = jnp.reshape(rows, (TOTAL * EMBED_DIM // 128, 128))
    cker = pl.kernel(
        _c_body,
        out_type=jax.ShapeDtypeStruct((SEQ, EMBED_DIM, BATCH), jnp.float32),
        mesh=mesh,
        compiler_params=pltpu.CompilerParams(use_tc_tiling_on_sc=True, needs_layout_passes=False, disable_bounds_checks=True),
        scratch_types=[
            pltpu.VMEM((128, 128), jnp.float32),
            pltpu.VMEM((128, 128), jnp.float32),
            pltpu.VMEM((EMBED_DIM, CCH + 16), jnp.float32),
            pltpu.VMEM((EMBED_DIM, CCH + 16), jnp.float32),
            pltpu.SemaphoreType.DMA,
            pltpu.SemaphoreType.DMA,
            pltpu.SemaphoreType.DMA,
            pltpu.SemaphoreType.DMA,
        ],
    )
    outp = cker(rows2)
    return jnp.transpose(outp, (2, 0, 1))
